# baseline (speedup 1.0000x reference)
import functools
import jax
import jax.numpy as jnp
from jax import lax
from jax.experimental import pallas as pl
from jax.experimental.pallas import tpu as pltpu
from jax.experimental.pallas import tpu_sc as plsc

_N = 50000
_E = 800000
_G = 64
_NE = 100000
_EMB = 64
_L = 3
_H = 2
_NAUG = _N + _G
_NPAD = 50176
_EA = _E + 2 * _N + _NAUG
_EAP = 950272
_E0 = _E + 2 * _N
_E0P = 901120
_NEP = 114688

_NW = 32


@functools.lru_cache(maxsize=1)
def _mesh():
    return plsc.VectorSubcoreMesh(core_axis_name="c", subcore_axis_name="s")


def _wid():
    return lax.axis_index("s") * 2 + lax.axis_index("c")


def _sc_gather(table, idx, D):
    B = idx.shape[0]
    bw = B // _NW
    CH = 512 if bw % 512 == 0 else (256 if bw % 256 == 0 else 64)
    assert bw % CH == 0

    @functools.partial(
        pl.kernel, mesh=_mesh(),
        compiler_params=pltpu.CompilerParams(use_tc_tiling_on_sc=False),
        out_type=jax.ShapeDtypeStruct((B, D), jnp.float32),
        scratch_types=[
            pltpu.VMEM((CH,), jnp.int32),
            pltpu.VMEM((CH, D), jnp.float32),
            pltpu.SemaphoreType.DMA,
        ],
    )
    def k(table_hbm, idx_hbm, out_hbm, idx_v, rows_v, sem):
        base = _wid() * bw

        def body(i, _):
            o = base + i * CH
            pltpu.sync_copy(idx_hbm.at[pl.ds(o, CH)], idx_v)
            pltpu.async_copy(table_hbm.at[idx_v], rows_v, sem).wait()
            pltpu.sync_copy(rows_v, out_hbm.at[pl.ds(o, CH)])
            return ()

        lax.fori_loop(0, bw // CH, body, ())

    return k(table, idx)


def _sc_scatter_add(vals, idx, n_out):
    B, Fc = vals.shape
    bw = B // _NW
    CH = 256 if bw % 256 == 0 else 64
    assert bw % CH == 0 and n_out % _NW == 0
    nw_rows = n_out // _NW

    @functools.partial(
        pl.kernel, mesh=_mesh(),
        compiler_params=pltpu.CompilerParams(use_tc_tiling_on_sc=False),
        out_type=jax.ShapeDtypeStruct((2, n_out, Fc), jnp.float32),
        scratch_types=[
            pltpu.VMEM((CH,), jnp.int32),
            pltpu.VMEM((CH, Fc), jnp.float32),
            pltpu.VMEM_SHARED((n_out, Fc), jnp.float32),
        ],
    )
    def k(vals_hbm, idx_hbm, zer_hbm, out_hbm, idx_v, vals_v, acc_sh):
        c = lax.axis_index("c")
        s = lax.axis_index("s")
        w = _wid()
        pltpu.sync_copy(zer_hbm.at[pl.ds(s * nw_rows, nw_rows)],
                        acc_sh.at[pl.ds(s * nw_rows, nw_rows)])
        plsc.subcore_barrier()

        base = w * bw

        def body(i, _):
            o = base + i * CH
            pltpu.sync_copy(idx_hbm.at[pl.ds(o, CH)], idx_v)
            pltpu.sync_copy(vals_hbm.at[pl.ds(o, CH)], vals_v)
            pltpu.sync_copy(vals_v, acc_sh.at[idx_v], add=True)
            return ()

        lax.fori_loop(0, bw // CH, body, ())
        plsc.subcore_barrier()
        pltpu.sync_copy(acc_sh.at[pl.ds(s * nw_rows, nw_rows)],
                        out_hbm.at[c, pl.ds(s * nw_rows, nw_rows)])

    zer = jnp.zeros((n_out, Fc), jnp.float32)
    return k(vals, idx, zer)


def _vgather(a, idx):
    dn = lax.GatherDimensionNumbers(offset_dims=(), collapsed_slice_dims=(0,),
                                    start_index_map=(0,))
    return lax.gather(a, idx[:, None], dn, slice_sizes=(1,),
                      mode=lax.GatherScatterMode.PROMISE_IN_BOUNDS)


def _seg_scan(a, d, prev_d, prev_a, combine):
    io = lax.iota(jnp.int32, 16)
    a = jnp.where(d == prev_d, combine(a, prev_a), a)
    for st in (1, 2, 4, 8):
        sh = jnp.maximum(io - st, 0)
        a_s = _vgather(a, sh)
        d_s = _vgather(d, sh)
        ok = (d_s == d) & (io >= st)
        a = jnp.where(ok, combine(a, a_s), a)
    return a


def _splat15(v):
    return _vgather(v, jnp.full((16,), 15, jnp.int32))


def _runend_mask(d, io):
    nxt = _vgather(d, jnp.minimum(io + 1, 15))
    return (d != nxt) | (io == 15)


def _sc_softmax(alphaT, sdst):
    CH = 4096
    n_ch = _EAP // CH

    @functools.partial(
        pl.kernel, mesh=_mesh(),
        compiler_params=pltpu.CompilerParams(use_tc_tiling_on_sc=False,
                                             needs_layout_passes=False),
        out_type=jax.ShapeDtypeStruct((_H, _EAP), jnp.float32),
        scratch_types=[
            pltpu.VMEM((CH,), jnp.float32),
            pltpu.VMEM((CH,), jnp.int32),
            pltpu.VMEM((CH,), jnp.float32),
            pltpu.VMEM((_NPAD,), jnp.float32),
            pltpu.VMEM((_NPAD,), jnp.float32),
        ],
    )
    def k(a_hbm, d_hbm, out_hbm, a_v, d_v, o_v, m_acc, s_acc):
        h = lax.axis_index("c")
        s = lax.axis_index("s")
        io = lax.iota(jnp.int32, 16)

        @pl.when(s == 0)
        def _():
            def p1(ci, carry):
                prev_d, prev_a = carry
                pltpu.sync_copy(a_hbm.at[h, pl.ds(ci * CH, CH)], a_v)
                pltpu.sync_copy(d_hbm.at[pl.ds(ci * CH, CH)], d_v)

                def inner(j, carry2):
                    pd, pa = carry2
                    a = a_v[pl.ds(j * 16, 16)]
                    d = d_v[pl.ds(j * 16, 16)]
                    a = _seg_scan(a, d, pd, pa, jnp.maximum)
                    plsc.store_scatter(m_acc, [d], a, mask=_runend_mask(d, io))
                    return (_splat15(d), _splat15(a))

                return lax.fori_loop(0, CH // 16, inner, (prev_d, prev_a))

            lax.fori_loop(0, n_ch, p1,
                          (jnp.full((16,), -1, jnp.int32),
                           jnp.zeros((16,), jnp.float32)))

            def p2(ci, carry):
                prev_d, prev_a = carry
                pltpu.sync_copy(a_hbm.at[h, pl.ds(ci * CH, CH)], a_v)
                pltpu.sync_copy(d_hbm.at[pl.ds(ci * CH, CH)], d_v)

                def inner(j, carry2):
                    pd, pa = carry2
                    a = a_v[pl.ds(j * 16, 16)]
                    d = d_v[pl.ds(j * 16, 16)]
                    m = plsc.load_gather(m_acc, [d])
                    e = jnp.exp(a - m)
                    e = _seg_scan(e, d, pd, pa, jnp.add)
                    plsc.store_scatter(s_acc, [d], e, mask=_runend_mask(d, io))
                    return (_splat15(d), _splat15(e))

                return lax.fori_loop(0, CH // 16, inner, (prev_d, prev_a))

            lax.fori_loop(0, n_ch, p2,
                          (jnp.full((16,), -1, jnp.int32),
                           jnp.zeros((16,), jnp.float32)))

            def p3(ci, _):
                pltpu.sync_copy(a_hbm.at[h, pl.ds(ci * CH, CH)], a_v)
                pltpu.sync_copy(d_hbm.at[pl.ds(ci * CH, CH)], d_v)

                def inner(j, __):
                    a = a_v[pl.ds(j * 16, 16)]
                    d = d_v[pl.ds(j * 16, 16)]
                    m = plsc.load_gather(m_acc, [d])
                    sm = plsc.load_gather(s_acc, [d])
                    o_v[pl.ds(j * 16, 16)] = jnp.exp(a - m) / (sm + 1e-16)
                    return ()

                lax.fori_loop(0, CH // 16, inner, ())
                pltpu.sync_copy(o_v, out_hbm.at[h, pl.ds(ci * CH, CH)])
                return ()

            lax.fori_loop(0, n_ch, p3, ())

    return k(alphaT, sdst)


def _pad_rows(a, n):
    return jnp.pad(a, ((0, n - a.shape[0]),) + ((0, 0),) * (a.ndim - 1))


def _merge(p):
    return p[0] + p[1]


def kernel(x, edge_attr, cond, edge_index, batch, non_edge_index, gen_W, gen_b,
           q_W, q_b, k_W, k_b, v_W, v_b, e_W, skip_W, skip_b, lin_W, lin_b,
           ff_W1, ff_b1, ff_W2, ff_b2):
    n_aug = _NAUG
    ei = edge_index.astype(jnp.int32)
    batch = batch.astype(jnp.int32)
    u = jnp.arange(_N, dtype=jnp.int32)
    v = batch + _N
    src0 = jnp.concatenate([ei[0], u, v])
    dst0 = jnp.concatenate([ei[1], v, u])
    loop_idx = jnp.arange(n_aug, dtype=jnp.int32)
    src_all = jnp.concatenate([src0, loop_idx])
    dst_all = jnp.concatenate([dst0, loop_idx])

    perm = jnp.argsort(dst_all).astype(jnp.int32)
    sdst = dst_all[perm]
    ssrc = src_all[perm]
    sdst_p = jnp.concatenate(
        [sdst, jnp.full((_EAP - _EA,), _NPAD - 1, jnp.int32)])
    ssrc_p = jnp.concatenate([ssrc, jnp.zeros((_EAP - _EA,), jnp.int32)])
    perm_p = jnp.concatenate([perm, jnp.zeros((_EAP - _EA,), jnp.int32)])

    e_p = jnp.zeros((2 * _N, _EMB), jnp.float32).at[:, 0].set(1.0)
    e0 = jnp.concatenate([edge_attr, e_p], 0)
    e0p = _pad_rows(e0, _E0P)
    d0p = jnp.concatenate(
        [dst0, jnp.full((_E0P - _E0,), _NPAD - 1, jnp.int32)])
    ones0 = jnp.zeros((_E0P, 8), jnp.float32).at[:_E0, 0].set(1.0)
    cnt0 = _merge(_sc_scatter_add(ones0, d0p, _NPAD))[:n_aug, 0]
    s_lo = _merge(_sc_scatter_add(e0p[:, :32], d0p, _NPAD))[:n_aug]
    s_hi = _merge(_sc_scatter_add(e0p[:, 32:], d0p, _NPAD))[:n_aug]
    loop_attr = jnp.concatenate([s_lo, s_hi], 1) / jnp.maximum(cnt0, 1.0)[:, None]

    e_all = jnp.concatenate([e0, loop_attr], 0)
    se = _sc_gather(_pad_rows(e_all, _EAP), perm_p, _EMB)

    aug_batch = jnp.concatenate([batch, jnp.arange(_G, dtype=jnp.int32)], 0)
    onehot = (aug_batch[:, None] == jnp.arange(_G)[None, :]).astype(jnp.float32)
    cnt_b = onehot.sum(0)

    def graph_ln(t):
        f = t.shape[-1]
        s1 = (onehot.T @ t).sum(-1)
        s2 = (onehot.T @ (t * t)).sum(-1)
        c = jnp.maximum(cnt_b, 1.0) * f
        mean = s1 / c
        var = s2 / c - mean * mean
        mn = onehot @ mean
        vn = onehot @ var
        return (t - mn[:, None]) / jnp.sqrt(vn + 1e-5)[:, None]

    h = jnp.concatenate([x, cond], 0)
    for l in range(_L):
        xn = graph_ln(h)
        xn_pad = _pad_rows(xn, _NPAD)
        gx = _sc_gather(xn_pad, ssrc_p, _EMB)[:_EA]
        msg = jax.nn.relu(gx + se[:_EA]) + 1e-7
        msgp = _pad_rows(msg, _EAP)
        a_lo = _merge(_sc_scatter_add(msgp[:, :32], sdst_p, _NPAD))[:n_aug]
        a_hi = _merge(_sc_scatter_add(msgp[:, 32:], sdst_p, _NPAD))[:n_aug]
        agg0 = jnp.concatenate([a_lo, a_hi], 1)
        agg = (agg0 + xn) @ gen_W[l] + gen_b[l]
        xin = jnp.concatenate([xn, agg], 1)
        q = xin @ q_W[l] + q_b[l]
        k = xin @ k_W[l] + k_b[l]
        vv = xin @ v_W[l] + v_b[l]
        sk = xin @ skip_W[l] + skip_b[l]
        eh = se[:_EA] @ e_W[l]
        qd = _sc_gather(_pad_rows(q, _NPAD), sdst_p, _H * _EMB)[:_EA]
        ks = _sc_gather(_pad_rows(k, _NPAD), ssrc_p, _H * _EMB)[:_EA]
        vs = _sc_gather(_pad_rows(vv, _NPAD), ssrc_p, _H * _EMB)[:_EA]
        t = (ks + eh).reshape(_EA, _H, _EMB)
        alpha = (qd.reshape(_EA, _H, _EMB) * t).sum(-1) / 8.0
        alphaT = jnp.pad(alpha.T, ((0, 0), (0, _EAP - _EA)))
        wT = _sc_softmax(alphaT, sdst_p)
        w = wT[:, :_EA].T
        wv = (vs.reshape(_EA, _H, _EMB) + eh.reshape(_EA, _H, _EMB)) * w[:, :, None]
        wv = wv.reshape(_EA, _H * _EMB)
        wvp = _pad_rows(wv, _EAP)
        o0 = _merge(_sc_scatter_add(wvp[:, 0:32], sdst_p, _NPAD))[:n_aug]
        o1 = _merge(_sc_scatter_add(wvp[:, 32:64], sdst_p, _NPAD))[:n_aug]
        o2 = _merge(_sc_scatter_add(wvp[:, 64:96], sdst_p, _NPAD))[:n_aug]
        o3 = _merge(_sc_scatter_add(wvp[:, 96:128], sdst_p, _NPAD))[:n_aug]
        out = jnp.concatenate([o0, o1, o2, o3], 1) + sk
        lh = out @ lin_W[l] + lin_b[l]
        z = graph_ln(lh)
        z = jax.nn.leaky_relu(z @ ff_W1[l] + ff_b1[l], 0.01)
        z = z @ ff_W2[l] + ff_b2[l]
        h = h + z

    n_emb = h[:_N]
    v_emb = h[_N:]
    cntg = jnp.maximum(cnt_b, 1.0)
    glob = (onehot[:_N].T @ n_emb) / cntg[:, None] + v_emb
    nep = non_edge_index.astype(jnp.int32)
    zpad = jnp.zeros((_NEP - _NE,), jnp.int32)
    ne0 = _sc_gather(_pad_rows(n_emb, _NPAD),
                     jnp.concatenate([nep[0], zpad]), _EMB)[:_NE]
    ne1 = _sc_gather(_pad_rows(n_emb, _NPAD),
                     jnp.concatenate([nep[1], zpad]), _EMB)[:_NE]
    ne_emb = ne0 + ne1
    return n_emb, glob, ne_emb


# speedup vs baseline: 1.3083x; 1.3083x over previous
import functools
import jax
import jax.numpy as jnp
from jax import lax
from jax.experimental import pallas as pl
from jax.experimental.pallas import tpu as pltpu
from jax.experimental.pallas import tpu_sc as plsc

_N = 50000
_E = 800000
_G = 64
_NE = 100000
_EMB = 64
_L = 3
_H = 2
_NAUG = _N + _G
_NPAD = 50176
_EA = _E + 2 * _N + _NAUG
_EAP = 950272
_E0 = _E + 2 * _N
_E0P = 901120
_NEP = 114688

_NW = 32


@functools.lru_cache(maxsize=1)
def _mesh():
    return plsc.VectorSubcoreMesh(core_axis_name="c", subcore_axis_name="s")


def _wid():
    return lax.axis_index("s") * 2 + lax.axis_index("c")


def _sc_gather(table, idx, D):
    B = idx.shape[0]
    bw = B // _NW
    CH = 512 if bw % 512 == 0 else (256 if bw % 256 == 0 else 64)
    assert bw % CH == 0

    @functools.partial(
        pl.kernel, mesh=_mesh(),
        compiler_params=pltpu.CompilerParams(use_tc_tiling_on_sc=False),
        out_type=jax.ShapeDtypeStruct((B, D), jnp.float32),
        scratch_types=[
            pltpu.VMEM((CH,), jnp.int32),
            pltpu.VMEM((CH, D), jnp.float32),
            pltpu.SemaphoreType.DMA,
        ],
    )
    def k(table_hbm, idx_hbm, out_hbm, idx_v, rows_v, sem):
        base = _wid() * bw

        def body(i, _):
            o = base + i * CH
            pltpu.sync_copy(idx_hbm.at[pl.ds(o, CH)], idx_v)
            pltpu.async_copy(table_hbm.at[idx_v], rows_v, sem).wait()
            pltpu.sync_copy(rows_v, out_hbm.at[pl.ds(o, CH)])
            return ()

        lax.fori_loop(0, bw // CH, body, ())

    return k(table, idx)


def _sc_scatter_add(vals, idx, n_out):
    B, Fc = vals.shape
    bw = B // _NW
    CH = 256 if bw % 256 == 0 else 64
    assert bw % CH == 0 and n_out % _NW == 0
    nw_rows = n_out // _NW

    @functools.partial(
        pl.kernel, mesh=_mesh(),
        compiler_params=pltpu.CompilerParams(use_tc_tiling_on_sc=False),
        out_type=jax.ShapeDtypeStruct((2, n_out, Fc), jnp.float32),
        scratch_types=[
            pltpu.VMEM((CH,), jnp.int32),
            pltpu.VMEM((CH, Fc), jnp.float32),
            pltpu.VMEM_SHARED((n_out, Fc), jnp.float32),
        ],
    )
    def k(vals_hbm, idx_hbm, zer_hbm, out_hbm, idx_v, vals_v, acc_sh):
        c = lax.axis_index("c")
        s = lax.axis_index("s")
        w = _wid()
        pltpu.sync_copy(zer_hbm.at[pl.ds(s * nw_rows, nw_rows)],
                        acc_sh.at[pl.ds(s * nw_rows, nw_rows)])
        plsc.subcore_barrier()

        base = w * bw

        def body(i, _):
            o = base + i * CH
            pltpu.sync_copy(idx_hbm.at[pl.ds(o, CH)], idx_v)
            pltpu.sync_copy(vals_hbm.at[pl.ds(o, CH)], vals_v)
            pltpu.sync_copy(vals_v, acc_sh.at[idx_v], add=True)
            return ()

        lax.fori_loop(0, bw // CH, body, ())
        plsc.subcore_barrier()
        pltpu.sync_copy(acc_sh.at[pl.ds(s * nw_rows, nw_rows)],
                        out_hbm.at[c, pl.ds(s * nw_rows, nw_rows)])

    zer = jnp.zeros((n_out, Fc), jnp.float32)
    return k(vals, idx, zer)


def _vgather(a, idx):
    dn = lax.GatherDimensionNumbers(offset_dims=(), collapsed_slice_dims=(0,),
                                    start_index_map=(0,))
    return lax.gather(a, idx[:, None], dn, slice_sizes=(1,),
                      mode=lax.GatherScatterMode.PROMISE_IN_BOUNDS)


def _seg_scan(a, d, prev_d, prev_a, combine):
    io = lax.iota(jnp.int32, 16)
    a = jnp.where(d == prev_d, combine(a, prev_a), a)
    for st in (1, 2, 4, 8):
        sh = jnp.maximum(io - st, 0)
        a_s = _vgather(a, sh)
        d_s = _vgather(d, sh)
        ok = (d_s == d) & (io >= st)
        a = jnp.where(ok, combine(a, a_s), a)
    return a


def _splat15(v):
    return _vgather(v, jnp.full((16,), 15, jnp.int32))


def _runend_mask(d, io):
    nxt = _vgather(d, jnp.minimum(io + 1, 15))
    return (d != nxt) | (io == 15)


def _allreduce16(v, combine):
    io = lax.iota(jnp.int32, 16)
    for st in (1, 2, 4, 8):
        v = combine(v, _vgather(v, io ^ st))
    return v


def _sc_softmax(alphaT, sdst):
    CH = 2048
    W = _EAP // 16
    n_ch = W // CH
    NEG = -3e38

    @functools.partial(
        pl.kernel, mesh=_mesh(),
        compiler_params=pltpu.CompilerParams(use_tc_tiling_on_sc=False,
                                             needs_layout_passes=False),
        out_type=jax.ShapeDtypeStruct((_H, _EAP), jnp.float32),
        scratch_types=[
            pltpu.VMEM((CH,), jnp.float32),
            pltpu.VMEM((CH,), jnp.int32),
            pltpu.VMEM((CH,), jnp.float32),
            pltpu.VMEM((_NPAD,), jnp.float32),
            pltpu.VMEM((_NPAD,), jnp.float32),
            pltpu.VMEM((16,), jnp.int32),
            pltpu.VMEM((16,), jnp.float32),
            pltpu.VMEM((16, 16), jnp.int32),
            pltpu.VMEM((16, 16), jnp.float32),
            pltpu.VMEM_SHARED((16, 16), jnp.int32),
            pltpu.VMEM_SHARED((16, 16), jnp.float32),
            pltpu.VMEM_SHARED((16, 16), jnp.int32),
            pltpu.VMEM_SHARED((16, 16), jnp.float32),
        ],
    )
    def k(a_hbm, d_hbm, out_hbm, a_v, d_v, o_v, m_acc, s_acc,
          st_i, st_f, pi_m, pf_m, fd_sh, fv_sh, ld_sh, lv_sh):
        h = lax.axis_index("c")
        s = lax.axis_index("s")
        io = lax.iota(jnp.int32, 16)
        base = s * W

        pltpu.sync_copy(d_hbm.at[pl.ds(base, 16)], st_i)
        fd = _vgather(st_i[...], jnp.zeros((16,), jnp.int32))

        def stream_pass(fn, carry0):
            def outer(ci, carry):
                pltpu.sync_copy(a_hbm.at[h, pl.ds(base + ci * CH, CH)], a_v)
                pltpu.sync_copy(d_hbm.at[pl.ds(base + ci * CH, CH)], d_v)

                def inner(j, c2):
                    a = a_v[pl.ds(j * 16, 16)]
                    d = d_v[pl.ds(j * 16, 16)]
                    return fn(a, d, c2)

                return lax.fori_loop(0, CH // 16, inner, carry)

            return lax.fori_loop(0, n_ch, outer, carry0)

        def publish(first_v, last_d, last_v):
            st_i[...] = fd
            pltpu.sync_copy(st_i, fd_sh.at[s])
            st_f[...] = first_v
            pltpu.sync_copy(st_f, fv_sh.at[s])
            st_i[...] = last_d
            pltpu.sync_copy(st_i, ld_sh.at[s])
            st_f[...] = last_v
            pltpu.sync_copy(st_f, lv_sh.at[s])

        z16 = jnp.zeros((16,), jnp.int32)

        def fixup(acc_ref, neutral, combine):
            pltpu.sync_copy(fd_sh, pi_m)
            fd_all = plsc.load_gather(pi_m, [io, z16])
            pltpu.sync_copy(ld_sh, pi_m)
            ld_all = plsc.load_gather(pi_m, [io, z16])
            pltpu.sync_copy(fv_sh, pf_m)
            fv_all = plsc.load_gather(pf_m, [io, z16])
            pltpu.sync_copy(lv_sh, pf_m)
            lv_all = plsc.load_gather(pf_m, [io, z16])
            for cand in (fd_all, ld_all):
                tot = jnp.full((16,), neutral, jnp.float32)
                for r in range(16):
                    idx = (io + r) & 15
                    rfd = _vgather(fd_all, idx)
                    rld = _vgather(ld_all, idx)
                    rfv = _vgather(fv_all, idx)
                    rlv = _vgather(lv_all, idx)
                    c = jnp.where(rfd == cand, rfv,
                                  jnp.where(rld == cand, rlv, neutral))
                    tot = combine(tot, c)
                plsc.store_scatter(acc_ref, [cand], tot)

        def p1(a, d, c):
            pd, pa, fm = c
            asc = _seg_scan(a, d, pd, pa, jnp.maximum)
            plsc.store_scatter(m_acc, [d], asc, mask=_runend_mask(d, io))
            fm = jnp.maximum(fm, jnp.where(d == fd, a, NEG))
            return (_splat15(d), _splat15(asc), fm)

        pd, pa, fm = stream_pass(
            p1, (jnp.full((16,), -1, jnp.int32),
                 jnp.full((16,), NEG, jnp.float32),
                 jnp.full((16,), NEG, jnp.float32)))
        publish(_allreduce16(fm, jnp.maximum), pd, pa)
        plsc.subcore_barrier()
        fixup(m_acc, NEG, jnp.maximum)
        plsc.subcore_barrier()

        def p2(a, d, c):
            pd2, pa2, fs = c
            m = plsc.load_gather(m_acc, [d])
            e = jnp.exp(a - m)
            esc = _seg_scan(e, d, pd2, pa2, jnp.add)
            plsc.store_scatter(s_acc, [d], esc, mask=_runend_mask(d, io))
            fs = fs + jnp.where(d == fd, e, 0.0)
            return (_splat15(d), _splat15(esc), fs)

        pd2, pa2, fs = stream_pass(
            p2, (jnp.full((16,), -1, jnp.int32),
                 jnp.zeros((16,), jnp.float32),
                 jnp.zeros((16,), jnp.float32)))
        publish(_allreduce16(fs, jnp.add), pd2, pa2)
        plsc.subcore_barrier()
        fixup(s_acc, 0.0, jnp.add)

        def p3(ci, _):
            pltpu.sync_copy(a_hbm.at[h, pl.ds(base + ci * CH, CH)], a_v)
            pltpu.sync_copy(d_hbm.at[pl.ds(base + ci * CH, CH)], d_v)

            def inner(j, __):
                a = a_v[pl.ds(j * 16, 16)]
                d = d_v[pl.ds(j * 16, 16)]
                m = plsc.load_gather(m_acc, [d])
                sm = plsc.load_gather(s_acc, [d])
                o_v[pl.ds(j * 16, 16)] = jnp.exp(a - m) / (sm + 1e-16)
                return ()

            lax.fori_loop(0, CH // 16, inner, ())
            pltpu.sync_copy(o_v, out_hbm.at[h, pl.ds(base + ci * CH, CH)])
            return ()

        lax.fori_loop(0, n_ch, p3, ())

    return k(alphaT, sdst)


def _pad_rows(a, n):
    return jnp.pad(a, ((0, n - a.shape[0]),) + ((0, 0),) * (a.ndim - 1))


def _merge(p):
    return p[0] + p[1]


def kernel(x, edge_attr, cond, edge_index, batch, non_edge_index, gen_W, gen_b,
           q_W, q_b, k_W, k_b, v_W, v_b, e_W, skip_W, skip_b, lin_W, lin_b,
           ff_W1, ff_b1, ff_W2, ff_b2):
    n_aug = _NAUG
    ei = edge_index.astype(jnp.int32)
    batch = batch.astype(jnp.int32)
    u = jnp.arange(_N, dtype=jnp.int32)
    v = batch + _N
    src0 = jnp.concatenate([ei[0], u, v])
    dst0 = jnp.concatenate([ei[1], v, u])
    loop_idx = jnp.arange(n_aug, dtype=jnp.int32)
    src_all = jnp.concatenate([src0, loop_idx])
    dst_all = jnp.concatenate([dst0, loop_idx])

    perm = jnp.argsort(dst_all).astype(jnp.int32)
    sdst = dst_all[perm]
    ssrc = src_all[perm]
    sdst_p = jnp.concatenate(
        [sdst, jnp.full((_EAP - _EA,), _NPAD - 1, jnp.int32)])
    ssrc_p = jnp.concatenate([ssrc, jnp.zeros((_EAP - _EA,), jnp.int32)])
    perm_p = jnp.concatenate([perm, jnp.zeros((_EAP - _EA,), jnp.int32)])

    e_p = jnp.zeros((2 * _N, _EMB), jnp.float32).at[:, 0].set(1.0)
    e0 = jnp.concatenate([edge_attr, e_p], 0)
    e0p = _pad_rows(e0, _E0P)
    d0p = jnp.concatenate(
        [dst0, jnp.full((_E0P - _E0,), _NPAD - 1, jnp.int32)])
    ones0 = jnp.zeros((_E0P, 8), jnp.float32).at[:_E0, 0].set(1.0)
    cnt0 = _merge(_sc_scatter_add(ones0, d0p, _NPAD))[:n_aug, 0]
    s_lo = _merge(_sc_scatter_add(e0p[:, :32], d0p, _NPAD))[:n_aug]
    s_hi = _merge(_sc_scatter_add(e0p[:, 32:], d0p, _NPAD))[:n_aug]
    loop_attr = jnp.concatenate([s_lo, s_hi], 1) / jnp.maximum(cnt0, 1.0)[:, None]

    e_all = jnp.concatenate([e0, loop_attr], 0)
    se = _sc_gather(_pad_rows(e_all, _EAP), perm_p, _EMB)

    aug_batch = jnp.concatenate([batch, jnp.arange(_G, dtype=jnp.int32)], 0)
    onehot = (aug_batch[:, None] == jnp.arange(_G)[None, :]).astype(jnp.float32)
    cnt_b = onehot.sum(0)

    def graph_ln(t):
        f = t.shape[-1]
        s1 = (onehot.T @ t).sum(-1)
        s2 = (onehot.T @ (t * t)).sum(-1)
        c = jnp.maximum(cnt_b, 1.0) * f
        mean = s1 / c
        var = s2 / c - mean * mean
        mn = onehot @ mean
        vn = onehot @ var
        return (t - mn[:, None]) / jnp.sqrt(vn + 1e-5)[:, None]

    h = jnp.concatenate([x, cond], 0)
    for l in range(_L):
        xn = graph_ln(h)
        xn_pad = _pad_rows(xn, _NPAD)
        gx = _sc_gather(xn_pad, ssrc_p, _EMB)[:_EA]
        msg = jax.nn.relu(gx + se[:_EA]) + 1e-7
        msgp = _pad_rows(msg, _EAP)
        a_lo = _merge(_sc_scatter_add(msgp[:, :32], sdst_p, _NPAD))[:n_aug]
        a_hi = _merge(_sc_scatter_add(msgp[:, 32:], sdst_p, _NPAD))[:n_aug]
        agg0 = jnp.concatenate([a_lo, a_hi], 1)
        agg = (agg0 + xn) @ gen_W[l] + gen_b[l]
        xin = jnp.concatenate([xn, agg], 1)
        q = xin @ q_W[l] + q_b[l]
        k = xin @ k_W[l] + k_b[l]
        vv = xin @ v_W[l] + v_b[l]
        sk = xin @ skip_W[l] + skip_b[l]
        eh = se[:_EA] @ e_W[l]
        qd = _sc_gather(_pad_rows(q, _NPAD), sdst_p, _H * _EMB)[:_EA]
        ks = _sc_gather(_pad_rows(k, _NPAD), ssrc_p, _H * _EMB)[:_EA]
        vs = _sc_gather(_pad_rows(vv, _NPAD), ssrc_p, _H * _EMB)[:_EA]
        t = (ks + eh).reshape(_EA, _H, _EMB)
        alpha = (qd.reshape(_EA, _H, _EMB) * t).sum(-1) / 8.0
        alphaT = jnp.pad(alpha.T, ((0, 0), (0, _EAP - _EA)))
        wT = _sc_softmax(alphaT, sdst_p)
        w = wT[:, :_EA].T
        wv = (vs.reshape(_EA, _H, _EMB) + eh.reshape(_EA, _H, _EMB)) * w[:, :, None]
        wv = wv.reshape(_EA, _H * _EMB)
        wvp = _pad_rows(wv, _EAP)
        o0 = _merge(_sc_scatter_add(wvp[:, 0:32], sdst_p, _NPAD))[:n_aug]
        o1 = _merge(_sc_scatter_add(wvp[:, 32:64], sdst_p, _NPAD))[:n_aug]
        o2 = _merge(_sc_scatter_add(wvp[:, 64:96], sdst_p, _NPAD))[:n_aug]
        o3 = _merge(_sc_scatter_add(wvp[:, 96:128], sdst_p, _NPAD))[:n_aug]
        out = jnp.concatenate([o0, o1, o2, o3], 1) + sk
        lh = out @ lin_W[l] + lin_b[l]
        z = graph_ln(lh)
        z = jax.nn.leaky_relu(z @ ff_W1[l] + ff_b1[l], 0.01)
        z = z @ ff_W2[l] + ff_b2[l]
        h = h + z

    n_emb = h[:_N]
    v_emb = h[_N:]
    cntg = jnp.maximum(cnt_b, 1.0)
    glob = (onehot[:_N].T @ n_emb) / cntg[:, None] + v_emb
    nep = non_edge_index.astype(jnp.int32)
    zpad = jnp.zeros((_NEP - _NE,), jnp.int32)
    ne0 = _sc_gather(_pad_rows(n_emb, _NPAD),
                     jnp.concatenate([nep[0], zpad]), _EMB)[:_NE]
    ne1 = _sc_gather(_pad_rows(n_emb, _NPAD),
                     jnp.concatenate([nep[1], zpad]), _EMB)[:_NE]
    ne_emb = ne0 + ne1
    return n_emb, glob, ne_emb
